# Initial kernel scaffold; baseline (speedup 1.0000x reference)
import jax
import jax.numpy as jnp
from jax.experimental import pallas as pl


def kernel(visual_feat, text_feat, pre_w1, pre_b1, pre_w2, pre_b2, pre_w3, pre_b3, ft_w, ft_b, fp_w, fp_b, agg_w, agg_b, q_w, q_b, k_w, k_b, v_w, v_b, in_w, in_b, out_w, out_b, post_w1, post_b1, post_w2, post_b2, post_w3, post_b3):
    raise NotImplementedError("write your pallas kernel here")


# speedup vs baseline: 1.3286x; 1.3286x over previous
import jax
import jax.numpy as jnp
from jax.experimental import pallas as pl
from jax.experimental.pallas import tpu as pltpu

_TOPK_RATIO = 0.1
_NH = 4
_BISECT_ITERS = 13


def _fold_kernel(q_w, q_b, k_w, k_b, v_w, v_b, in_w, in_b, out_w, out_b,
                 pw1, pb1, wq_o, bq_o, wk_o, bk_o, wv_o, bv_o, wop_o, bop_o):
    E = q_w.shape[1]
    dot = lambda a, b: jax.lax.dot(a, b, preferred_element_type=jnp.float32)
    in_q = in_w[:, :E]
    in_k = in_w[:, E:2 * E]
    in_v = in_w[:, 2 * E:]
    wq_o[...] = dot(q_w[...], in_q)
    bq_o[...] = dot(q_b[...], in_q) + in_b[:, :E]
    wk_o[...] = dot(k_w[...], in_k)
    bk_o[...] = dot(k_b[...], in_k) + in_b[:, E:2 * E]
    wv_o[...] = dot(v_w[...], in_v)
    bv_o[...] = dot(v_b[...], in_v) + in_b[:, 2 * E:]
    wop_o[...] = dot(out_w[...], pw1[...])
    bop_o[...] = dot(out_b[...], pw1[...]) + pb1[...]


def _main_kernel(v_ref, t_ref,
                 pre_w1, pre_b1, pre_w2, pre_b2, pre_w3t, pre_b3,
                 ft_w, ft_b, fp_w, fp_b, agg_wt, agg_b,
                 wq, bq, wk, bk, wv, bv,
                 wop, bop, post_w2, post_b2, post_w3t, post_b3,
                 out_ref):
    f32 = jnp.float32
    dot = lambda a, b: jax.lax.dot(a, b, preferred_element_type=f32)
    dot_nt = lambda a, b: jax.lax.dot_general(
        a, b, (((1,), (1,)), ((), ())), preferred_element_type=f32)
    relu = lambda x: jnp.maximum(x, 0.0)

    vis = v_ref[0]
    txt = t_ref[0]
    S, VD = vis.shape
    E = wq.shape[1]
    HD = E // _NH
    k_sel = min(max(1, int(_TOPK_RATIO * S)), S)

    h1 = relu(dot(vis, pre_w1[...]) + pre_b1[...])
    h2 = relu(dot(h1, pre_w2[...]) + pre_b2[...])
    scores = dot_nt(pre_w3t[...], h2) + pre_b3[...]

    kk = dot(vis, wk[:VD]) + dot(txt, wk[VD:]) + bk[...]
    vv = dot(vis, wv[:VD]) + dot(txt, wv[VD:]) + bv[...]

    lo = jnp.min(scores, axis=1, keepdims=True)
    hi = jnp.max(scores, axis=1, keepdims=True)
    wfrac = jax.lax.broadcasted_iota(jnp.int32, (8, 1), 0).astype(f32) * 0.125
    for _ in range(_BISECT_ITERS):
        ts = lo + (hi - lo) * wfrac
        cnt = jnp.sum((scores >= ts).astype(f32), axis=1, keepdims=True)
        ok = cnt >= f32(k_sel)
        lo = jnp.max(jnp.where(ok, ts, -jnp.inf), axis=0, keepdims=True)
        hi = jnp.minimum(
            jnp.min(jnp.where(ok, jnp.inf, ts), axis=0, keepdims=True), hi)
    sel = scores >= lo

    tt = relu(dot(vis, ft_w[...]) + ft_b[...])
    c = relu(dot(tt, fp_w[...]) + fp_b[...])
    lg = dot_nt(agg_wt[...], c) + agg_b[...]
    lg = jnp.where(sel, lg, -jnp.inf)
    lg = lg - jnp.max(lg, axis=1, keepdims=True)
    e = jnp.exp(lg)
    aw = e / jnp.sum(e, axis=1, keepdims=True)
    cam = dot(aw, c)

    q = dot(cam, wq[...]) + bq[...]
    bmt = (jax.lax.broadcasted_iota(jnp.int32, (_NH, E), 1) // HD ==
           jax.lax.broadcasted_iota(jnp.int32, (_NH, E), 0)).astype(f32)
    logits = dot_nt(kk * q, bmt) * (1.0 / jnp.sqrt(f32(HD)))
    logits = logits - jnp.max(logits, axis=0, keepdims=True)
    ee = jnp.exp(logits)
    probs = ee / jnp.sum(ee, axis=0, keepdims=True)
    pe = dot(probs, bmt)
    ctx = jnp.sum(pe * vv, axis=0, keepdims=True)

    g1 = relu(dot(ctx, wop[...]) + bop[...])
    g2 = relu(dot(g1, post_w2[...]) + post_b2[...])
    o = jnp.sum(g2 * post_w3t[...], axis=1, keepdims=True) + post_b3[...]
    out_ref[0] = jnp.broadcast_to(o, (1, S))


def kernel(visual_feat, text_feat,
           pre_w1, pre_b1, pre_w2, pre_b2, pre_w3, pre_b3,
           ft_w, ft_b, fp_w, fp_b, agg_w, agg_b,
           q_w, q_b, k_w, k_b, v_w, v_b, in_w, in_b, out_w, out_b,
           post_w1, post_b1, post_w2, post_b2, post_w3, post_b3):
    B, S, VD = visual_feat.shape
    TD = text_feat.shape[2]
    E = q_w.shape[1]
    CAM = fp_w.shape[1]
    FD = VD + TD
    row = lambda x: x.reshape(1, -1)

    wq, bq, wk, bk, wv, bv, wop, bop = pl.pallas_call(
        _fold_kernel,
        out_shape=[
            jax.ShapeDtypeStruct((CAM, E), jnp.float32),
            jax.ShapeDtypeStruct((1, E), jnp.float32),
            jax.ShapeDtypeStruct((FD, E), jnp.float32),
            jax.ShapeDtypeStruct((1, E), jnp.float32),
            jax.ShapeDtypeStruct((FD, E), jnp.float32),
            jax.ShapeDtypeStruct((1, E), jnp.float32),
            jax.ShapeDtypeStruct((E, post_w1.shape[1]), jnp.float32),
            jax.ShapeDtypeStruct((1, post_w1.shape[1]), jnp.float32),
        ],
    )(q_w, row(q_b), k_w, row(k_b), v_w, row(v_b), in_w, row(in_b),
      out_w, row(out_b), post_w1, row(post_b1))

    full = lambda a: pl.BlockSpec(a.shape, lambda b: (0,) * a.ndim)
    operands = (
        visual_feat, text_feat,
        pre_w1, row(pre_b1), pre_w2, row(pre_b2), row(pre_w3), row(pre_b3),
        ft_w, row(ft_b), fp_w, row(fp_b), row(agg_w), row(agg_b),
        wq, bq, wk, bk, wv, bv,
        wop, bop, post_w2, row(post_b2), row(post_w3), row(post_b3),
    )
    in_specs = [
        pl.BlockSpec((1, S, VD), lambda b: (b, 0, 0)),
        pl.BlockSpec((1, S, TD), lambda b: (b, 0, 0)),
    ] + [full(a) for a in operands[2:]]

    out3 = pl.pallas_call(
        _main_kernel,
        grid=(B,),
        in_specs=in_specs,
        out_specs=pl.BlockSpec((1, 1, S), lambda b: (b, 0, 0)),
        out_shape=jax.ShapeDtypeStruct((B, 1, S), jnp.float32),
        compiler_params=pltpu.CompilerParams(
            dimension_semantics=("parallel",),
            vmem_limit_bytes=52 * 1024 * 1024,
        ),
    )(*operands)
    return out3.reshape(B, S)
